# baseline (speedup 1.0000x reference)
import dataclasses
import functools

import jax
import jax.numpy as jnp
from jax import lax
from jax.experimental import pallas as pl
from jax.experimental.pallas import tpu as pltpu
from jax.experimental.pallas import tpu_sc as plsc

N = 1024
B = 8
NH = 4
F = 128
NW = 32
RPW = N // NW
CAP = RPW * N
L = 16

_mesh = plsc.VectorSubcoreMesh(core_axis_name="c", subcore_axis_name="s")

_cp = pltpu.CompilerParams()
if "needs_layout_passes" in pltpu.CompilerParams.__dataclass_fields__:
    _cp = dataclasses.replace(_cp, needs_layout_passes=False)

NEG = -3e38


def _splat(x, dtype=jnp.int32):
    return jnp.full((L,), x, dtype)


def _adj_body(emb_ref, adj_ref):
    emb = emb_ref[...]
    sq = jnp.sum(emb * emb, axis=1, keepdims=True)
    nrm = emb * lax.rsqrt(jnp.maximum(sq, 1e-12))
    adj_ref[...] = lax.dot_general(
        nrm, nrm, (((1,), (1,)), ((), ())), preferred_element_type=jnp.float32
    )


def _mm(a, b):
    return lax.dot_general(a, b, (((1,), (0,)), ((), ())),
                           preferred_element_type=jnp.float32)


def _proj1_body(x_ref, wp_ref, bp_ref, w1_ref, a1s_ref, a1d_ref,
                h_ref, st_ref, d_ref):
    xp = _mm(x_ref[0], wp_ref[...]) + bp_ref[...]
    h = _mm(xp, w1_ref[...])
    h_ref[0] = h
    st_ref[0] = lax.dot_general(a1s_ref[...], h, (((0,), (1,)), ((), ())),
                                preferred_element_type=jnp.float32)
    d_ref[0] = lax.dot_general(a1d_ref[...], h, (((0,), (1,)), ((), ())),
                               preferred_element_type=jnp.float32)


def _proj2_body(acc_ref, den_ref, e_ref, b1_ref, w2_ref, a2s_ref, a2d_ref,
                h_ref, st_ref, d_ref):
    denx = lax.dot_general(den_ref[0], e_ref[...], (((0,), (0,)), ((), ())),
                           preferred_element_type=jnp.float32)
    o1 = jnp.maximum(acc_ref[0] / denx + b1_ref[...], 0.0)
    h = _mm(o1, w2_ref[...])
    h_ref[0] = h
    st_ref[0] = lax.dot_general(a2s_ref[...], h, (((0,), (1,)), ((), ())),
                                preferred_element_type=jnp.float32)
    d_ref[0] = lax.dot_general(a2d_ref[...], h, (((0,), (1,)), ((), ())),
                               preferred_element_type=jnp.float32)


def _final_body(acc_ref, den_ref, e_ref, b2_ref, out_ref):
    denx = lax.dot_general(den_ref[0], e_ref[...], (((0,), (0,)), ((), ())),
                           preferred_element_type=jnp.float32)
    out_ref[0] = acc_ref[0] / denx + b2_ref[...]


@functools.partial(
    pl.kernel,
    out_type=(
        jax.ShapeDtypeStruct((NW, CAP + L), jnp.int32),
        jax.ShapeDtypeStruct((NW, CAP + L), jnp.int32),
        jax.ShapeDtypeStruct((N,), jnp.int32),
    ),
    mesh=_mesh,
    scratch_types=[
        pltpu.VMEM((RPW, N), jnp.float32),
        pltpu.VMEM((CAP + L,), jnp.int32),
        pltpu.VMEM((CAP + L,), jnp.int32),
        pltpu.VMEM((RPW,), jnp.int32),
        pltpu.SemaphoreType.DMA,
    ],
    compiler_params=_cp,
)
def _scan_kernel(adj_hbm, cols_hbm, rows_hbm, counts_hbm,
                 strip_v, cols_v, rows_v, cnt_v, sem):
    w = lax.axis_index("s") * 2 + lax.axis_index("c")
    base = w * RPW
    pltpu.async_copy(adj_hbm.at[pl.ds(base, RPW)], strip_v, sem).wait()
    iota = lax.iota(jnp.int32, L)

    def row_body(r, pos):
        def chunk(c, p):
            v = strip_v[r, pl.ds(c * L, L)]
            m = v > 0.5
            plsc.store_compressed(cols_v.at[pl.ds(p, L)], iota + c * L, mask=m)
            plsc.store_compressed(rows_v.at[pl.ds(p, L)],
                                  jnp.full((L,), r, jnp.int32), mask=m)
            return p + lax.reduce_max(plsc.all_reduce_population_count(m), (0,))

        pos2 = lax.fori_loop(0, N // L, chunk, pos)
        plsc.store_scatter(cnt_v, [iota * 0 + r],
                           jnp.full((L,), pos2 - pos, jnp.int32), mask=iota == 0)
        return pos2

    total = lax.fori_loop(0, RPW, row_body, jnp.int32(0))
    pltpu.sync_copy(cnt_v, counts_hbm.at[pl.ds(base, RPW)])

    def wb(i, _):
        pltpu.sync_copy(cols_v.at[pl.ds(i * 1024, 1024)],
                        cols_hbm.at[w, pl.ds(i * 1024, 1024)])
        pltpu.sync_copy(rows_v.at[pl.ds(i * 1024, 1024)],
                        rows_hbm.at[w, pl.ds(i * 1024, 1024)])
        return 0

    lax.fori_loop(0, (total + 1023) // 1024, wb, 0)


@functools.partial(
    pl.kernel,
    out_type=(
        jax.ShapeDtypeStruct((B, N, F), jnp.float32),
        jax.ShapeDtypeStruct((N, B * NH), jnp.float32),
    ),
    mesh=_mesh,
    scratch_types=[
        pltpu.VMEM((RPW,), jnp.int32),
        pltpu.VMEM((B * NH * N,), jnp.float32),
        pltpu.VMEM((B * NH, N), jnp.float32),
        pltpu.VMEM((RPW, B * NH), jnp.float32),
        pltpu.VMEM((RPW, B * NH), jnp.float32),
        pltpu.VMEM((B, RPW, F), jnp.float32),
        pltpu.VMEM((L,), jnp.int32),
        pltpu.VMEM((L,), jnp.int32),
        pltpu.VMEM((B * L,), jnp.int32),
        pltpu.VMEM((B * L, F), jnp.float32),
        pltpu.VMEM((B * NH, L), jnp.float32),
        pltpu.SemaphoreType.DMA,
        pltpu.SemaphoreType.DMA,
    ],
    compiler_params=_cp,
)
def _agg_kernel(h_hbm, s_hbm, d_hbm, cols_hbm, rows_hbm, counts_hbm, z_hbm,
                acc_hbm, den_hbm,
                cnt_v, s_v, d_v, m_v, den_v, acc_v,
                colw_v, roww_v, idx_v, hrow_v, w_v, sem, zsem):
    w = lax.axis_index("s") * 2 + lax.axis_index("c")
    base = w * RPW
    iota = lax.iota(jnp.int32, L)
    zeros = jnp.zeros((L,), jnp.float32)
    negs = jnp.full((L,), NEG, jnp.float32)

    zcp = pltpu.make_async_copy(z_hbm, acc_v, zsem)
    zcp.start()

    pltpu.sync_copy(counts_hbm.at[pl.ds(base, RPW)], cnt_v)
    pltpu.sync_copy(s_hbm, s_v)
    pltpu.sync_copy(d_hbm, d_v)

    total = lax.reduce_sum(cnt_v[pl.ds(0, L)] + cnt_v[pl.ds(L, L)], (0,))
    nwin = (total + L - 1) // L

    def zrow(r, _):
        m_v[r, pl.ds(0, L)] = negs
        m_v[r, pl.ds(L, L)] = negs
        den_v[r, pl.ds(0, L)] = zeros
        den_v[r, pl.ds(L, L)] = zeros
        return 0

    lax.fori_loop(0, RPW, zrow, 0)

    def winA(win, _):
        pltpu.sync_copy(cols_hbm.at[w, pl.ds(win * L, L)], colw_v)
        pltpu.sync_copy(rows_hbm.at[w, pl.ds(win * L, L)], roww_v)

        def edge(k, _):
            jq = plsc.load_gather(colw_v, [_splat(k)])
            r_e = jnp.max(plsc.load_gather(roww_v, [_splat(k)]))
            s0 = plsc.load_gather(s_v, [iota * N + jq])
            s1 = plsc.load_gather(s_v, [(iota + L) * N + jq])
            m_v[r_e, pl.ds(0, L)] = jnp.maximum(m_v[r_e, pl.ds(0, L)], s0)
            m_v[r_e, pl.ds(L, L)] = jnp.maximum(m_v[r_e, pl.ds(L, L)], s1)
            return 0

        lax.fori_loop(0, jnp.minimum(total - win * L, L), edge, 0)
        return 0

    lax.fori_loop(0, nwin, winA, 0)

    def finrow(r, _):
        for half in range(2):
            dh = plsc.load_gather(d_v, [iota + half * L, _splat(base + r)])
            a = dh + m_v[r, pl.ds(half * L, L)]
            m_v[r, pl.ds(half * L, L)] = jnp.where(a > 0, a, 0.2 * a)
        return 0

    lax.fori_loop(0, RPW, finrow, 0)
    zcp.wait()

    def winB(win, _):
        pltpu.sync_copy(cols_hbm.at[w, pl.ds(win * L, L)], colw_v)
        pltpu.sync_copy(rows_hbm.at[w, pl.ds(win * L, L)], roww_v)
        lm = iota + win * L < total
        jv = jnp.where(lm, colw_v[...], 0)
        rv = jnp.where(lm, roww_v[...], 0)
        for b in range(B):
            idx_v[pl.ds(b * L, L)] = jv + b * N
        cp = pltpu.make_async_copy(h_hbm.at[idx_v], hrow_v, sem)
        cp.start()
        for c in range(B * NH):
            sh = plsc.load_gather(s_v, [c * N + jv])
            dh = plsc.load_gather(d_v, [_splat(c), base + rv])
            mh = plsc.load_gather(m_v, [rv, _splat(c)])
            ev = dh + sh
            ev = jnp.where(ev > 0, ev, 0.2 * ev)
            wh = jnp.exp(ev - mh)
            w_v[c, :] = jnp.where(lm, wh, 0.0)
        cp.wait()

        def edge(k, _):
            r_e = jnp.max(plsc.load_gather(roww_v, [_splat(k)]))
            wc0 = plsc.load_gather(w_v, [iota, _splat(k)])
            wc1 = plsc.load_gather(w_v, [iota + L, _splat(k)])
            plsc.addupdate(den_v.at[r_e, pl.ds(0, L)], wc0)
            plsc.addupdate(den_v.at[r_e, pl.ds(L, L)], wc1)
            for b in range(B):
                wks = [plsc.load_gather(w_v, [_splat(b * NH + h), _splat(k)])
                       for h in range(NH)]
                for c in range(F // L):
                    plsc.addupdate(
                        acc_v.at[b, r_e, pl.ds(c * L, L)],
                        wks[c // 2] * hrow_v[b * L + k, pl.ds(c * L, L)])
            return 0

        lax.fori_loop(0, jnp.minimum(total - win * L, L), edge, 0)
        return 0

    lax.fori_loop(0, nwin, winB, 0)

    for b in range(B):
        pltpu.sync_copy(acc_v.at[b], acc_hbm.at[b, pl.ds(base, RPW)])
    pltpu.sync_copy(den_v, den_hbm.at[pl.ds(base, RPW)])


def _blockdiag(a):
    heads, dim = a.shape
    eye = jnp.eye(heads, dtype=a.dtype)
    return (a[:, :, None] * eye[:, None, :]).reshape(heads * dim, heads)


def kernel(x, embedding, W_proj, b_proj, W1, a1_src, a1_dst, b1, W2,
           a2_src, a2_dst, b2):
    adj = pl.pallas_call(
        _adj_body,
        out_shape=jax.ShapeDtypeStruct((N, N), jnp.float32),
    )(embedding)

    cols, rows, counts = _scan_kernel(adj)

    w1r = W1.reshape(F, F)
    w2r = W2.reshape(F, F)
    a1s = _blockdiag(a1_src)
    a1d = _blockdiag(a1_dst)
    a2s = _blockdiag(a2_src)
    a2d = _blockdiag(a2_dst)
    bp2 = b_proj.reshape(1, F)
    b1r = b1.reshape(1, F)
    b2r = b2.reshape(1, F)
    hexp = _blockdiag(jnp.ones((NH, F // NH), jnp.float32)).T

    full = lambda shape: pl.BlockSpec(shape, lambda b: (0,) * len(shape))
    batched = lambda shape: pl.BlockSpec((1,) + shape,
                                         lambda b: (b,) + (0,) * len(shape))
    h1, st1, d1 = pl.pallas_call(
        _proj1_body,
        grid=(B,),
        in_specs=[batched((N, F)), full((F, F)), full((1, F)), full((F, F)),
                  full((F, NH)), full((F, NH))],
        out_specs=[batched((N, F)), batched((NH, N)), batched((NH, N))],
        out_shape=[
            jax.ShapeDtypeStruct((B, N, F), jnp.float32),
            jax.ShapeDtypeStruct((B, NH, N), jnp.float32),
            jax.ShapeDtypeStruct((B, NH, N), jnp.float32),
        ],
    )(x, W_proj, bp2, w1r, a1s, a1d)

    zstrip = jnp.zeros((B, RPW, F), jnp.float32)
    acc1, den1 = _agg_kernel(h1.reshape(B * N, F), st1.reshape(B * NH * N),
                             d1.reshape(B * NH, N), cols, rows, counts,
                             zstrip)

    h2, st2, d2 = pl.pallas_call(
        _proj2_body,
        grid=(B,),
        in_specs=[batched((N, F)), batched((NH, N)), full((NH, F)),
                  full((1, F)), full((F, F)), full((F, NH)), full((F, NH))],
        out_specs=[batched((N, F)), batched((NH, N)), batched((NH, N))],
        out_shape=[
            jax.ShapeDtypeStruct((B, N, F), jnp.float32),
            jax.ShapeDtypeStruct((B, NH, N), jnp.float32),
            jax.ShapeDtypeStruct((B, NH, N), jnp.float32),
        ],
    )(acc1, jnp.transpose(den1.reshape(N, B, NH), (1, 2, 0)), hexp, b1r,
      w2r, a2s, a2d)

    acc2, den2 = _agg_kernel(h2.reshape(B * N, F), st2.reshape(B * NH * N),
                             d2.reshape(B * NH, N), cols, rows, counts,
                             zstrip)

    out = pl.pallas_call(
        _final_body,
        grid=(B,),
        in_specs=[batched((N, F)), batched((NH, N)), full((NH, F)),
                  full((1, F))],
        out_specs=batched((N, F)),
        out_shape=jax.ShapeDtypeStruct((B, N, F), jnp.float32),
    )(acc2, jnp.transpose(den2.reshape(N, B, NH), (1, 2, 0)), hexp, b2r)
    return out


# speedup vs baseline: 1.0250x; 1.0250x over previous
import dataclasses
import functools

import jax
import jax.numpy as jnp
from jax import lax
from jax.experimental import pallas as pl
from jax.experimental.pallas import tpu as pltpu
from jax.experimental.pallas import tpu_sc as plsc

N = 1024
B = 8
NH = 4
F = 128
NW = 32
RPW = N // NW
CAP = RPW * N
L = 16

_mesh = plsc.VectorSubcoreMesh(core_axis_name="c", subcore_axis_name="s")

_cp = pltpu.CompilerParams()
if "needs_layout_passes" in pltpu.CompilerParams.__dataclass_fields__:
    _cp = dataclasses.replace(_cp, needs_layout_passes=False)

NEG = -3e38


def _splat(x, dtype=jnp.int32):
    return jnp.full((L,), x, dtype)


def _adj_body(emb_ref, adj_ref):
    emb = emb_ref[...]
    sq = jnp.sum(emb * emb, axis=1, keepdims=True)
    nrm = emb * lax.rsqrt(jnp.maximum(sq, 1e-12))
    adj_ref[...] = lax.dot_general(
        nrm, nrm, (((1,), (1,)), ((), ())), preferred_element_type=jnp.float32
    )


def _mm(a, b):
    return lax.dot_general(a, b, (((1,), (0,)), ((), ())),
                           preferred_element_type=jnp.float32)


def _proj1_body(x_ref, wp_ref, bp_ref, w1_ref, a1s_ref, a1d_ref,
                h_ref, st_ref, d_ref):
    xp = _mm(x_ref[0], wp_ref[...]) + bp_ref[...]
    h = _mm(xp, w1_ref[...])
    h_ref[0] = h
    st_ref[0] = lax.dot_general(a1s_ref[...], h, (((0,), (1,)), ((), ())),
                                preferred_element_type=jnp.float32)
    d_ref[0] = _mm(h, a1d_ref[...])


def _proj2_body(acc_ref, den_ref, e_ref, b1_ref, w2_ref, a2s_ref, a2d_ref,
                h_ref, st_ref, d_ref):
    denx = lax.dot_general(den_ref[0], e_ref[...], (((0,), (0,)), ((), ())),
                           preferred_element_type=jnp.float32)
    o1 = jnp.maximum(acc_ref[0] / denx + b1_ref[...], 0.0)
    h = _mm(o1, w2_ref[...])
    h_ref[0] = h
    st_ref[0] = lax.dot_general(a2s_ref[...], h, (((0,), (1,)), ((), ())),
                                preferred_element_type=jnp.float32)
    d_ref[0] = _mm(h, a2d_ref[...])


def _final_body(acc_ref, den_ref, e_ref, b2_ref, out_ref):
    denx = lax.dot_general(den_ref[0], e_ref[...], (((0,), (0,)), ((), ())),
                           preferred_element_type=jnp.float32)
    out_ref[0] = acc_ref[0] / denx + b2_ref[...]


@functools.partial(
    pl.kernel,
    out_type=(
        jax.ShapeDtypeStruct((NW, CAP + L), jnp.int32),
        jax.ShapeDtypeStruct((NW, CAP + L), jnp.int32),
        jax.ShapeDtypeStruct((N,), jnp.int32),
    ),
    mesh=_mesh,
    scratch_types=[
        pltpu.VMEM((RPW, N), jnp.float32),
        pltpu.VMEM((CAP + L,), jnp.int32),
        pltpu.VMEM((CAP + L,), jnp.int32),
        pltpu.VMEM((RPW,), jnp.int32),
        pltpu.SemaphoreType.DMA,
    ],
    compiler_params=_cp,
)
def _scan_kernel(adj_hbm, cols_hbm, rows_hbm, counts_hbm,
                 strip_v, cols_v, rows_v, cnt_v, sem):
    w = lax.axis_index("s") * 2 + lax.axis_index("c")
    base = w * RPW
    pltpu.async_copy(adj_hbm.at[pl.ds(base, RPW)], strip_v, sem).wait()
    iota = lax.iota(jnp.int32, L)

    def row_body(r, pos):
        def chunk(c, p):
            v = strip_v[r, pl.ds(c * L, L)]
            m = v > 0.5
            plsc.store_compressed(cols_v.at[pl.ds(p, L)], iota + c * L, mask=m)
            plsc.store_compressed(rows_v.at[pl.ds(p, L)],
                                  jnp.full((L,), r, jnp.int32), mask=m)
            return p + lax.reduce_max(plsc.all_reduce_population_count(m), (0,))

        pos2 = lax.fori_loop(0, N // L, chunk, pos)
        plsc.store_scatter(cnt_v, [iota * 0 + r],
                           jnp.full((L,), pos2 - pos, jnp.int32), mask=iota == 0)
        return pos2

    total = lax.fori_loop(0, RPW, row_body, jnp.int32(0))
    pltpu.sync_copy(cnt_v, counts_hbm.at[pl.ds(base, RPW)])

    def wb(i, _):
        pltpu.sync_copy(cols_v.at[pl.ds(i * 1024, 1024)],
                        cols_hbm.at[w, pl.ds(i * 1024, 1024)])
        pltpu.sync_copy(rows_v.at[pl.ds(i * 1024, 1024)],
                        rows_hbm.at[w, pl.ds(i * 1024, 1024)])
        return 0

    lax.fori_loop(0, (total + 1023) // 1024, wb, 0)


@functools.partial(
    pl.kernel,
    out_type=(
        jax.ShapeDtypeStruct((B, N, F), jnp.float32),
        jax.ShapeDtypeStruct((N, B * NH), jnp.float32),
    ),
    mesh=_mesh,
    scratch_types=[
        pltpu.VMEM((RPW,), jnp.int32),
        pltpu.VMEM((B * NH * N,), jnp.float32),
        pltpu.VMEM((B, RPW, NH), jnp.float32),
        pltpu.VMEM((RPW, B * NH), jnp.float32),
        pltpu.VMEM((RPW, B * NH), jnp.float32),
        pltpu.VMEM((B, RPW, F), jnp.float32),
        pltpu.VMEM((L,), jnp.int32),
        pltpu.VMEM((L,), jnp.int32),
        pltpu.VMEM((B * L,), jnp.int32),
        pltpu.VMEM((B * L, F), jnp.float32),
        pltpu.VMEM((B * NH, L), jnp.float32),
        pltpu.SemaphoreType.DMA,
        pltpu.SemaphoreType.DMA,
    ],
    compiler_params=_cp,
)
def _agg_kernel(h_hbm, s_hbm, d_hbm, cols_hbm, rows_hbm, counts_hbm, z_hbm,
                acc_hbm, den_hbm,
                cnt_v, s_v, d_v, m_v, den_v, acc_v,
                colw_v, roww_v, idx_v, hrow_v, w_v, sem, zsem):
    w = lax.axis_index("s") * 2 + lax.axis_index("c")
    base = w * RPW
    iota = lax.iota(jnp.int32, L)
    zeros = jnp.zeros((L,), jnp.float32)
    negs = jnp.full((L,), NEG, jnp.float32)

    zcp = pltpu.make_async_copy(z_hbm, acc_v, zsem)
    zcp.start()

    pltpu.sync_copy(counts_hbm.at[pl.ds(base, RPW)], cnt_v)
    pltpu.sync_copy(s_hbm, s_v)
    pltpu.sync_copy(d_hbm.at[:, pl.ds(base, RPW)], d_v)

    total = lax.reduce_sum(cnt_v[pl.ds(0, L)] + cnt_v[pl.ds(L, L)], (0,))
    nwin = (total + L - 1) // L

    def zrow(r, _):
        m_v[r, pl.ds(0, L)] = negs
        m_v[r, pl.ds(L, L)] = negs
        den_v[r, pl.ds(0, L)] = zeros
        den_v[r, pl.ds(L, L)] = zeros
        return 0

    lax.fori_loop(0, RPW, zrow, 0)

    def winA(win, _):
        pltpu.sync_copy(cols_hbm.at[w, pl.ds(win * L, L)], colw_v)
        pltpu.sync_copy(rows_hbm.at[w, pl.ds(win * L, L)], roww_v)

        def edge(k, _):
            jq = plsc.load_gather(colw_v, [_splat(k)])
            r_e = jnp.max(plsc.load_gather(roww_v, [_splat(k)]))
            s0 = plsc.load_gather(s_v, [iota * N + jq])
            s1 = plsc.load_gather(s_v, [(iota + L) * N + jq])
            m_v[r_e, pl.ds(0, L)] = jnp.maximum(m_v[r_e, pl.ds(0, L)], s0)
            m_v[r_e, pl.ds(L, L)] = jnp.maximum(m_v[r_e, pl.ds(L, L)], s1)
            return 0

        lax.fori_loop(0, jnp.minimum(total - win * L, L), edge, 0)
        return 0

    lax.fori_loop(0, nwin, winA, 0)

    def finrow(r, _):
        for half in range(2):
            cc = iota + half * L
            dh = plsc.load_gather(d_v, [cc // NH, _splat(r), cc % NH])
            a = dh + m_v[r, pl.ds(half * L, L)]
            m_v[r, pl.ds(half * L, L)] = jnp.where(a > 0, a, 0.2 * a)
        return 0

    lax.fori_loop(0, RPW, finrow, 0)
    zcp.wait()

    def winB(win, _):
        pltpu.sync_copy(cols_hbm.at[w, pl.ds(win * L, L)], colw_v)
        pltpu.sync_copy(rows_hbm.at[w, pl.ds(win * L, L)], roww_v)
        lm = iota + win * L < total
        jv = jnp.where(lm, colw_v[...], 0)
        rv = jnp.where(lm, roww_v[...], 0)
        for b in range(B):
            idx_v[pl.ds(b * L, L)] = jv + b * N
        cp = pltpu.make_async_copy(h_hbm.at[idx_v], hrow_v, sem)
        cp.start()
        for c in range(B * NH):
            sh = plsc.load_gather(s_v, [c * N + jv])
            dh = plsc.load_gather(d_v, [_splat(c // NH), rv, _splat(c % NH)])
            mh = plsc.load_gather(m_v, [rv, _splat(c)])
            ev = dh + sh
            ev = jnp.where(ev > 0, ev, 0.2 * ev)
            wh = jnp.exp(ev - mh)
            w_v[c, :] = jnp.where(lm, wh, 0.0)
        cp.wait()

        def edge(k, _):
            r_e = jnp.max(plsc.load_gather(roww_v, [_splat(k)]))
            wc0 = plsc.load_gather(w_v, [iota, _splat(k)])
            wc1 = plsc.load_gather(w_v, [iota + L, _splat(k)])
            plsc.addupdate(den_v.at[r_e, pl.ds(0, L)], wc0)
            plsc.addupdate(den_v.at[r_e, pl.ds(L, L)], wc1)
            for b in range(B):
                wks = [plsc.load_gather(w_v, [_splat(b * NH + h), _splat(k)])
                       for h in range(NH)]
                for c in range(F // L):
                    plsc.addupdate(
                        acc_v.at[b, r_e, pl.ds(c * L, L)],
                        wks[c // 2] * hrow_v[b * L + k, pl.ds(c * L, L)])
            return 0

        lax.fori_loop(0, jnp.minimum(total - win * L, L), edge, 0)
        return 0

    lax.fori_loop(0, nwin, winB, 0)

    for b in range(B):
        pltpu.sync_copy(acc_v.at[b], acc_hbm.at[b, pl.ds(base, RPW)])
    pltpu.sync_copy(den_v, den_hbm.at[pl.ds(base, RPW)])


def _blockdiag(a):
    heads, dim = a.shape
    eye = jnp.eye(heads, dtype=a.dtype)
    return (a[:, :, None] * eye[:, None, :]).reshape(heads * dim, heads)


def kernel(x, embedding, W_proj, b_proj, W1, a1_src, a1_dst, b1, W2,
           a2_src, a2_dst, b2):
    adj = pl.pallas_call(
        _adj_body,
        out_shape=jax.ShapeDtypeStruct((N, N), jnp.float32),
    )(embedding)

    cols, rows, counts = _scan_kernel(adj)

    w1r = W1.reshape(F, F)
    w2r = W2.reshape(F, F)
    a1s = _blockdiag(a1_src)
    a1d = _blockdiag(a1_dst)
    a2s = _blockdiag(a2_src)
    a2d = _blockdiag(a2_dst)
    bp2 = b_proj.reshape(1, F)
    b1r = b1.reshape(1, F)
    b2r = b2.reshape(1, F)
    hexp = _blockdiag(jnp.ones((NH, F // NH), jnp.float32)).T

    full = lambda shape: pl.BlockSpec(shape, lambda b: (0,) * len(shape))
    batched = lambda shape: pl.BlockSpec((1,) + shape,
                                         lambda b: (b,) + (0,) * len(shape))
    h1, st1, d1 = pl.pallas_call(
        _proj1_body,
        grid=(B,),
        in_specs=[batched((N, F)), full((F, F)), full((1, F)), full((F, F)),
                  full((F, NH)), full((F, NH))],
        out_specs=[batched((N, F)), batched((NH, N)), batched((N, NH))],
        out_shape=[
            jax.ShapeDtypeStruct((B, N, F), jnp.float32),
            jax.ShapeDtypeStruct((B, NH, N), jnp.float32),
            jax.ShapeDtypeStruct((B, N, NH), jnp.float32),
        ],
    )(x, W_proj, bp2, w1r, a1s, a1d)

    zstrip = jnp.zeros((B, RPW, F), jnp.float32)
    acc1, den1 = _agg_kernel(h1.reshape(B * N, F), st1.reshape(B * NH * N),
                             d1, cols, rows, counts, zstrip)

    h2, st2, d2 = pl.pallas_call(
        _proj2_body,
        grid=(B,),
        in_specs=[batched((N, F)), batched((NH, N)), full((NH, F)),
                  full((1, F)), full((F, F)), full((F, NH)), full((F, NH))],
        out_specs=[batched((N, F)), batched((NH, N)), batched((N, NH))],
        out_shape=[
            jax.ShapeDtypeStruct((B, N, F), jnp.float32),
            jax.ShapeDtypeStruct((B, NH, N), jnp.float32),
            jax.ShapeDtypeStruct((B, N, NH), jnp.float32),
        ],
    )(acc1, jnp.transpose(den1.reshape(N, B, NH), (1, 2, 0)), hexp, b1r,
      w2r, a2s, a2d)

    acc2, den2 = _agg_kernel(h2.reshape(B * N, F), st2.reshape(B * NH * N),
                             d2, cols, rows, counts, zstrip)

    out = pl.pallas_call(
        _final_body,
        grid=(B,),
        in_specs=[batched((N, F)), batched((NH, N)), full((NH, F)),
                  full((1, F))],
        out_specs=batched((N, F)),
        out_shape=jax.ShapeDtypeStruct((B, N, F), jnp.float32),
    )(acc2, jnp.transpose(den2.reshape(N, B, NH), (1, 2, 0)), hexp, b2r)
    return out


# speedup vs baseline: 1.0655x; 1.0396x over previous
import dataclasses
import functools

import jax
import jax.numpy as jnp
from jax import lax
from jax.experimental import pallas as pl
from jax.experimental.pallas import tpu as pltpu
from jax.experimental.pallas import tpu_sc as plsc

N = 1024
B = 8
NH = 4
F = 128
NW = 32
RPW = N // NW
CAP = RPW * N
L = 16

_mesh = plsc.VectorSubcoreMesh(core_axis_name="c", subcore_axis_name="s")

_cp = pltpu.CompilerParams()
if "needs_layout_passes" in pltpu.CompilerParams.__dataclass_fields__:
    _cp = dataclasses.replace(_cp, needs_layout_passes=False)

NEG = -3e38


def _splat(x, dtype=jnp.int32):
    return jnp.full((L,), x, dtype)


def _adj_body(emb_ref, adj_ref):
    emb = emb_ref[...]
    sq = jnp.sum(emb * emb, axis=1, keepdims=True)
    nrm = emb * lax.rsqrt(jnp.maximum(sq, 1e-12))
    adj_ref[...] = lax.dot_general(
        nrm, nrm, (((1,), (1,)), ((), ())), preferred_element_type=jnp.float32
    )


def _mm(a, b):
    return lax.dot_general(a, b, (((1,), (0,)), ((), ())),
                           preferred_element_type=jnp.float32)


def _proj1_body(x_ref, wp_ref, bp_ref, w1_ref, a1s_ref, a1d_ref,
                h_ref, st_ref, d_ref):
    xp = _mm(x_ref[0], wp_ref[...]) + bp_ref[...]
    h = _mm(xp, w1_ref[...])
    h_ref[0] = h
    st_ref[0] = lax.dot_general(a1s_ref[...], h, (((0,), (1,)), ((), ())),
                                preferred_element_type=jnp.float32)
    d_ref[0] = _mm(h, a1d_ref[...])


def _proj2_body(acc_ref, den_ref, e_ref, b1_ref, w2_ref, a2s_ref, a2d_ref,
                h_ref, st_ref, d_ref):
    denx = lax.dot_general(den_ref[0], e_ref[...], (((0,), (0,)), ((), ())),
                           preferred_element_type=jnp.float32)
    o1 = jnp.maximum(acc_ref[0] / denx + b1_ref[...], 0.0)
    h = _mm(o1, w2_ref[...])
    h_ref[0] = h
    st_ref[0] = lax.dot_general(a2s_ref[...], h, (((0,), (1,)), ((), ())),
                                preferred_element_type=jnp.float32)
    d_ref[0] = _mm(h, a2d_ref[...])


def _final_body(acc_ref, den_ref, e_ref, b2_ref, out_ref):
    denx = lax.dot_general(den_ref[0], e_ref[...], (((0,), (0,)), ((), ())),
                           preferred_element_type=jnp.float32)
    out_ref[0] = acc_ref[0] / denx + b2_ref[...]


@functools.partial(
    pl.kernel,
    out_type=(
        jax.ShapeDtypeStruct((NW, CAP + L), jnp.int32),
        jax.ShapeDtypeStruct((N,), jnp.int32),
    ),
    mesh=_mesh,
    scratch_types=[
        pltpu.VMEM((RPW, N), jnp.float32),
        pltpu.VMEM((CAP + L,), jnp.int32),
        pltpu.VMEM((RPW,), jnp.int32),
        pltpu.SemaphoreType.DMA,
    ],
    compiler_params=_cp,
)
def _scan_kernel(adj_hbm, cols_hbm, counts_hbm, strip_v, cols_v, cnt_v, sem):
    w = lax.axis_index("s") * 2 + lax.axis_index("c")
    base = w * RPW
    pltpu.async_copy(adj_hbm.at[pl.ds(base, RPW)], strip_v, sem).wait()
    iota = lax.iota(jnp.int32, L)

    def row_body(r, pos):
        def chunk(c, p):
            v = strip_v[r, pl.ds(c * L, L)]
            m = v > 0.5
            plsc.store_compressed(cols_v.at[pl.ds(p, L)],
                                  iota + c * L + r * N, mask=m)
            return p + lax.reduce_max(plsc.all_reduce_population_count(m), (0,))

        pos2 = lax.fori_loop(0, N // L, chunk, pos)
        plsc.store_scatter(cnt_v, [iota * 0 + r],
                           jnp.full((L,), pos2 - pos, jnp.int32), mask=iota == 0)
        return pos2

    total = lax.fori_loop(0, RPW, row_body, jnp.int32(0))
    pltpu.sync_copy(cnt_v, counts_hbm.at[pl.ds(base, RPW)])

    def wb(i, _):
        pltpu.sync_copy(cols_v.at[pl.ds(i * 1024, 1024)],
                        cols_hbm.at[w, pl.ds(i * 1024, 1024)])
        return 0

    lax.fori_loop(0, (total + 1023) // 1024, wb, 0)


@functools.partial(
    pl.kernel,
    out_type=(
        jax.ShapeDtypeStruct((B, N, F), jnp.float32),
        jax.ShapeDtypeStruct((N, B * NH), jnp.float32),
    ),
    mesh=_mesh,
    scratch_types=[
        pltpu.VMEM((RPW,), jnp.int32),
        pltpu.VMEM((B * NH * N,), jnp.float32),
        pltpu.VMEM((B, RPW, NH), jnp.float32),
        pltpu.VMEM((RPW, B * NH), jnp.float32),
        pltpu.VMEM((RPW, B * NH), jnp.float32),
        pltpu.VMEM((B, RPW, F), jnp.float32),
        pltpu.VMEM((L,), jnp.int32),
        pltpu.VMEM((L,), jnp.int32),
        pltpu.VMEM((B * L,), jnp.int32),
        pltpu.VMEM((B * L, F), jnp.float32),
        pltpu.VMEM((B * NH, L), jnp.float32),
        pltpu.SemaphoreType.DMA,
        pltpu.SemaphoreType.DMA,
    ],
    compiler_params=_cp,
)
def _agg_kernel(h_hbm, s_hbm, d_hbm, cols_hbm, counts_hbm, z_hbm,
                acc_hbm, den_hbm,
                cnt_v, s_v, d_v, m_v, den_v, acc_v,
                colw_v, roww_v, idx_v, hrow_v, w_v, sem, zsem):
    w = lax.axis_index("s") * 2 + lax.axis_index("c")
    base = w * RPW
    iota = lax.iota(jnp.int32, L)
    zeros = jnp.zeros((L,), jnp.float32)
    negs = jnp.full((L,), NEG, jnp.float32)

    zcp = pltpu.make_async_copy(z_hbm, acc_v, zsem)
    zcp.start()

    pltpu.sync_copy(counts_hbm.at[pl.ds(base, RPW)], cnt_v)
    pltpu.sync_copy(s_hbm, s_v)
    pltpu.sync_copy(d_hbm.at[:, pl.ds(base, RPW)], d_v)

    total = lax.reduce_sum(cnt_v[pl.ds(0, L)] + cnt_v[pl.ds(L, L)], (0,))
    nwin = (total + L - 1) // L

    def zrow(r, _):
        m_v[r, pl.ds(0, L)] = negs
        m_v[r, pl.ds(L, L)] = negs
        den_v[r, pl.ds(0, L)] = zeros
        den_v[r, pl.ds(L, L)] = zeros
        return 0

    lax.fori_loop(0, RPW, zrow, 0)

    def winA(win, _):
        pltpu.sync_copy(cols_hbm.at[w, pl.ds(win * L, L)], colw_v)
        pw = colw_v[...]
        colw_v[...] = pw & (N - 1)
        roww_v[...] = pw >> 10

        def edge(k, _):
            jq = plsc.load_gather(colw_v, [_splat(k)])
            r_e = jnp.max(plsc.load_gather(roww_v, [_splat(k)]))
            s0 = plsc.load_gather(s_v, [iota * N + jq])
            s1 = plsc.load_gather(s_v, [(iota + L) * N + jq])
            m_v[r_e, pl.ds(0, L)] = jnp.maximum(m_v[r_e, pl.ds(0, L)], s0)
            m_v[r_e, pl.ds(L, L)] = jnp.maximum(m_v[r_e, pl.ds(L, L)], s1)
            return 0

        lax.fori_loop(0, jnp.minimum(total - win * L, L), edge, 0)
        return 0

    lax.fori_loop(0, nwin, winA, 0)

    def finrow(r, _):
        for half in range(2):
            cc = iota + half * L
            dh = plsc.load_gather(d_v, [cc // NH, _splat(r), cc % NH])
            a = dh + m_v[r, pl.ds(half * L, L)]
            m_v[r, pl.ds(half * L, L)] = jnp.where(a > 0, a, 0.2 * a)
        return 0

    lax.fori_loop(0, RPW, finrow, 0)
    zcp.wait()

    def winB(win, _):
        pltpu.sync_copy(cols_hbm.at[w, pl.ds(win * L, L)], colw_v)
        pw = colw_v[...]
        lm = iota + win * L < total
        jv = jnp.where(lm, pw & (N - 1), 0)
        rv = jnp.where(lm, pw >> 10, 0)
        roww_v[...] = rv
        for b in range(B):
            idx_v[pl.ds(b * L, L)] = jv + b * N
        cp = pltpu.make_async_copy(h_hbm.at[idx_v], hrow_v, sem)
        cp.start()
        for c in range(B * NH):
            sh = plsc.load_gather(s_v, [c * N + jv])
            dh = plsc.load_gather(d_v, [_splat(c // NH), rv, _splat(c % NH)])
            mh = plsc.load_gather(m_v, [rv, _splat(c)])
            ev = dh + sh
            ev = jnp.where(ev > 0, ev, 0.2 * ev)
            wh = jnp.exp(ev - mh)
            w_v[c, :] = jnp.where(lm, wh, 0.0)
        cp.wait()

        def edge(k, _):
            r_e = jnp.max(plsc.load_gather(roww_v, [_splat(k)]))
            wc0 = plsc.load_gather(w_v, [iota, _splat(k)])
            wc1 = plsc.load_gather(w_v, [iota + L, _splat(k)])
            plsc.addupdate(den_v.at[r_e, pl.ds(0, L)], wc0)
            plsc.addupdate(den_v.at[r_e, pl.ds(L, L)], wc1)
            for b in range(B):
                wks = [plsc.load_gather(w_v, [_splat(b * NH + h), _splat(k)])
                       for h in range(NH)]
                for c in range(F // L):
                    plsc.addupdate(
                        acc_v.at[b, r_e, pl.ds(c * L, L)],
                        wks[c // 2] * hrow_v[b * L + k, pl.ds(c * L, L)])
            return 0

        lax.fori_loop(0, jnp.minimum(total - win * L, L), edge, 0)
        return 0

    lax.fori_loop(0, nwin, winB, 0)

    for b in range(B):
        pltpu.sync_copy(acc_v.at[b], acc_hbm.at[b, pl.ds(base, RPW)])
    pltpu.sync_copy(den_v, den_hbm.at[pl.ds(base, RPW)])


def _blockdiag(a):
    heads, dim = a.shape
    eye = jnp.eye(heads, dtype=a.dtype)
    return (a[:, :, None] * eye[:, None, :]).reshape(heads * dim, heads)


def kernel(x, embedding, W_proj, b_proj, W1, a1_src, a1_dst, b1, W2,
           a2_src, a2_dst, b2):
    adj = pl.pallas_call(
        _adj_body,
        out_shape=jax.ShapeDtypeStruct((N, N), jnp.float32),
    )(embedding)

    cols, counts = _scan_kernel(adj)

    w1r = W1.reshape(F, F)
    w2r = W2.reshape(F, F)
    a1s = _blockdiag(a1_src)
    a1d = _blockdiag(a1_dst)
    a2s = _blockdiag(a2_src)
    a2d = _blockdiag(a2_dst)
    bp2 = b_proj.reshape(1, F)
    b1r = b1.reshape(1, F)
    b2r = b2.reshape(1, F)
    hexp = _blockdiag(jnp.ones((NH, F // NH), jnp.float32)).T

    full = lambda shape: pl.BlockSpec(shape, lambda b: (0,) * len(shape))
    batched = lambda shape: pl.BlockSpec((1,) + shape,
                                         lambda b: (b,) + (0,) * len(shape))
    h1, st1, d1 = pl.pallas_call(
        _proj1_body,
        grid=(B,),
        in_specs=[batched((N, F)), full((F, F)), full((1, F)), full((F, F)),
                  full((F, NH)), full((F, NH))],
        out_specs=[batched((N, F)), batched((NH, N)), batched((N, NH))],
        out_shape=[
            jax.ShapeDtypeStruct((B, N, F), jnp.float32),
            jax.ShapeDtypeStruct((B, NH, N), jnp.float32),
            jax.ShapeDtypeStruct((B, N, NH), jnp.float32),
        ],
    )(x, W_proj, bp2, w1r, a1s, a1d)

    zstrip = jnp.zeros((B, RPW, F), jnp.float32)
    acc1, den1 = _agg_kernel(h1.reshape(B * N, F), st1.reshape(B * NH * N),
                             d1, cols, counts, zstrip)

    h2, st2, d2 = pl.pallas_call(
        _proj2_body,
        grid=(B,),
        in_specs=[batched((N, F)), batched((NH, N)), full((NH, F)),
                  full((1, F)), full((F, F)), full((F, NH)), full((F, NH))],
        out_specs=[batched((N, F)), batched((NH, N)), batched((N, NH))],
        out_shape=[
            jax.ShapeDtypeStruct((B, N, F), jnp.float32),
            jax.ShapeDtypeStruct((B, NH, N), jnp.float32),
            jax.ShapeDtypeStruct((B, N, NH), jnp.float32),
        ],
    )(acc1, jnp.transpose(den1.reshape(N, B, NH), (1, 2, 0)), hexp, b1r,
      w2r, a2s, a2d)

    acc2, den2 = _agg_kernel(h2.reshape(B * N, F), st2.reshape(B * NH * N),
                             d2, cols, counts, zstrip)

    out = pl.pallas_call(
        _final_body,
        grid=(B,),
        in_specs=[batched((N, F)), batched((NH, N)), full((NH, F)),
                  full((1, F))],
        out_specs=batched((N, F)),
        out_shape=jax.ShapeDtypeStruct((B, N, F), jnp.float32),
    )(acc2, jnp.transpose(den2.reshape(N, B, NH), (1, 2, 0)), hexp, b2r)
    return out


# speedup vs baseline: 1.0808x; 1.0143x over previous
import dataclasses
import functools

import jax
import jax.numpy as jnp
from jax import lax
from jax.experimental import pallas as pl
from jax.experimental.pallas import tpu as pltpu
from jax.experimental.pallas import tpu_sc as plsc

N = 1024
B = 8
NH = 4
F = 128
NW = 32
RPW = N // NW
CAP = RPW * N
L = 16

_mesh = plsc.VectorSubcoreMesh(core_axis_name="c", subcore_axis_name="s")

_cp = pltpu.CompilerParams()
if "needs_layout_passes" in pltpu.CompilerParams.__dataclass_fields__:
    _cp = dataclasses.replace(_cp, needs_layout_passes=False)

NEG = -3e38


def _splat(x, dtype=jnp.int32):
    return jnp.full((L,), x, dtype)


def _adj_body(emb_ref, adj_ref):
    emb = emb_ref[...]
    sq = jnp.sum(emb * emb, axis=1, keepdims=True)
    nrm = emb * lax.rsqrt(jnp.maximum(sq, 1e-12))
    adj_ref[...] = lax.dot_general(
        nrm, nrm, (((1,), (1,)), ((), ())), preferred_element_type=jnp.float32
    )


def _mm(a, b):
    return lax.dot_general(a, b, (((1,), (0,)), ((), ())),
                           preferred_element_type=jnp.float32)


def _proj1_body(x_ref, wp_ref, bp_ref, w1_ref, a1s_ref, a1d_ref,
                h_ref, st_ref, d_ref):
    xp = _mm(x_ref[0], wp_ref[...]) + bp_ref[...]
    h = _mm(xp, w1_ref[...])
    h_ref[0] = h
    st_ref[0] = lax.dot_general(a1s_ref[...], h, (((0,), (1,)), ((), ())),
                                preferred_element_type=jnp.float32)
    d_ref[0] = _mm(h, a1d_ref[...])


def _proj2_body(acc_ref, den_ref, e_ref, b1_ref, w2_ref, a2s_ref, a2d_ref,
                h_ref, st_ref, d_ref):
    denx = lax.dot_general(den_ref[0], e_ref[...], (((0,), (0,)), ((), ())),
                           preferred_element_type=jnp.float32)
    o1 = jnp.maximum(acc_ref[0] / denx + b1_ref[...], 0.0)
    h = _mm(o1, w2_ref[...])
    h_ref[0] = h
    st_ref[0] = lax.dot_general(a2s_ref[...], h, (((0,), (1,)), ((), ())),
                                preferred_element_type=jnp.float32)
    d_ref[0] = _mm(h, a2d_ref[...])


def _final_body(acc_ref, den_ref, e_ref, b2_ref, out_ref):
    denx = lax.dot_general(den_ref[0], e_ref[...], (((0,), (0,)), ((), ())),
                           preferred_element_type=jnp.float32)
    out_ref[0] = acc_ref[0] / denx + b2_ref[...]


@functools.partial(
    pl.kernel,
    out_type=(
        jax.ShapeDtypeStruct((NW, CAP + L), jnp.int32),
        jax.ShapeDtypeStruct((N,), jnp.int32),
    ),
    mesh=_mesh,
    scratch_types=[
        pltpu.VMEM((RPW, N), jnp.float32),
        pltpu.VMEM((CAP + L,), jnp.int32),
        pltpu.VMEM((RPW,), jnp.int32),
        pltpu.SemaphoreType.DMA,
    ],
    compiler_params=_cp,
)
def _scan_kernel(adj_hbm, cols_hbm, counts_hbm, strip_v, cols_v, cnt_v, sem):
    w = lax.axis_index("s") * 2 + lax.axis_index("c")
    base = w * RPW
    pltpu.async_copy(adj_hbm.at[pl.ds(base, RPW)], strip_v, sem).wait()
    iota = lax.iota(jnp.int32, L)

    def row_body(r, pos):
        def chunk(c, p):
            v = strip_v[r, pl.ds(c * L, L)]
            m = v > 0.5
            plsc.store_compressed(cols_v.at[pl.ds(p, L)],
                                  iota + c * L + r * N, mask=m)
            return p + lax.reduce_max(plsc.all_reduce_population_count(m), (0,))

        pos2 = lax.fori_loop(0, N // L, chunk, pos)
        plsc.store_scatter(cnt_v, [iota * 0 + r],
                           jnp.full((L,), pos2 - pos, jnp.int32), mask=iota == 0)
        return pos2

    total = lax.fori_loop(0, RPW, row_body, jnp.int32(0))
    pltpu.sync_copy(cnt_v, counts_hbm.at[pl.ds(base, RPW)])

    def wb(i, _):
        pltpu.sync_copy(cols_v.at[pl.ds(i * 1024, 1024)],
                        cols_hbm.at[w, pl.ds(i * 1024, 1024)])
        return 0

    lax.fori_loop(0, (total + 1023) // 1024, wb, 0)


@functools.partial(
    pl.kernel,
    out_type=(
        jax.ShapeDtypeStruct((B, N, F), jnp.float32),
        jax.ShapeDtypeStruct((N, B * NH), jnp.float32),
    ),
    mesh=_mesh,
    scratch_types=[
        pltpu.VMEM((RPW,), jnp.int32),
        pltpu.VMEM((B * NH * N,), jnp.float32),
        pltpu.VMEM((B, RPW, NH), jnp.float32),
        pltpu.VMEM((RPW, B * NH), jnp.float32),
        pltpu.VMEM((RPW, B * NH), jnp.float32),
        pltpu.VMEM((B, RPW, F), jnp.float32),
        pltpu.VMEM((L,), jnp.int32),
        pltpu.VMEM((L,), jnp.int32),
        pltpu.VMEM((B * L,), jnp.int32),
        pltpu.VMEM((B * L, F), jnp.float32),
        pltpu.VMEM((B * NH, L), jnp.float32),
        pltpu.SemaphoreType.DMA,
        pltpu.SemaphoreType.DMA,
    ],
    compiler_params=_cp,
)
def _agg_kernel(h_hbm, s_hbm, d_hbm, cols_hbm, counts_hbm, z_hbm,
                acc_hbm, den_hbm,
                cnt_v, s_v, d_v, m_v, den_v, acc_v,
                colw_v, roww_v, idx_v, hrow_v, w_v, sem, zsem):
    w = lax.axis_index("s") * 2 + lax.axis_index("c")
    base = w * RPW
    iota = lax.iota(jnp.int32, L)
    zeros = jnp.zeros((L,), jnp.float32)
    negs = jnp.full((L,), NEG, jnp.float32)

    zcp = pltpu.make_async_copy(z_hbm, acc_v, zsem)
    zcp.start()

    scp = pltpu.make_async_copy(s_hbm, s_v, sem)
    scp.start()
    pltpu.sync_copy(counts_hbm.at[pl.ds(base, RPW)], cnt_v)
    pltpu.sync_copy(d_hbm.at[:, pl.ds(base, RPW)], d_v)

    total = lax.reduce_sum(cnt_v[pl.ds(0, L)] + cnt_v[pl.ds(L, L)], (0,))
    nwin = (total + L - 1) // L

    def zrow(r, _):
        m_v[r, pl.ds(0, L)] = negs
        m_v[r, pl.ds(L, L)] = negs
        den_v[r, pl.ds(0, L)] = zeros
        den_v[r, pl.ds(L, L)] = zeros
        return 0

    lax.fori_loop(0, RPW, zrow, 0)
    scp.wait()

    def winA(win, _):
        pltpu.sync_copy(cols_hbm.at[w, pl.ds(win * L, L)], colw_v)
        pw = colw_v[...]
        colw_v[...] = pw & (N - 1)
        roww_v[...] = pw >> 10

        def edge(k, _):
            jq = plsc.load_gather(colw_v, [_splat(k)])
            r_e = jnp.max(plsc.load_gather(roww_v, [_splat(k)]))
            s0 = plsc.load_gather(s_v, [iota * N + jq])
            s1 = plsc.load_gather(s_v, [(iota + L) * N + jq])
            m_v[r_e, pl.ds(0, L)] = jnp.maximum(m_v[r_e, pl.ds(0, L)], s0)
            m_v[r_e, pl.ds(L, L)] = jnp.maximum(m_v[r_e, pl.ds(L, L)], s1)
            return 0

        lax.fori_loop(0, jnp.minimum(total - win * L, L), edge, 0)
        return 0

    lax.fori_loop(0, nwin, winA, 0)

    def finrow(r, _):
        for half in range(2):
            cc = iota + half * L
            dh = plsc.load_gather(d_v, [cc // NH, _splat(r), cc % NH])
            a = dh + m_v[r, pl.ds(half * L, L)]
            m_v[r, pl.ds(half * L, L)] = jnp.where(a > 0, a, 0.2 * a)
        return 0

    lax.fori_loop(0, RPW, finrow, 0)
    zcp.wait()

    def winB(win, _):
        pltpu.sync_copy(cols_hbm.at[w, pl.ds(win * L, L)], colw_v)
        pw = colw_v[...]
        lm = iota + win * L < total
        jv = jnp.where(lm, pw & (N - 1), 0)
        rv = jnp.where(lm, pw >> 10, 0)
        roww_v[...] = rv
        for b in range(B):
            idx_v[pl.ds(b * L, L)] = jv + b * N
        cp = pltpu.make_async_copy(h_hbm.at[idx_v], hrow_v, sem)
        cp.start()
        for c in range(B * NH):
            sh = plsc.load_gather(s_v, [c * N + jv])
            dh = plsc.load_gather(d_v, [_splat(c // NH), rv, _splat(c % NH)])
            mh = plsc.load_gather(m_v, [rv, _splat(c)])
            ev = dh + sh
            ev = jnp.where(ev > 0, ev, 0.2 * ev)
            wh = jnp.exp(ev - mh)
            w_v[c, :] = jnp.where(lm, wh, 0.0)
        cp.wait()

        def edge(k, _):
            r_e = jnp.max(plsc.load_gather(roww_v, [_splat(k)]))
            wc0 = plsc.load_gather(w_v, [iota, _splat(k)])
            wc1 = plsc.load_gather(w_v, [iota + L, _splat(k)])
            plsc.addupdate(den_v.at[r_e, pl.ds(0, L)], wc0)
            plsc.addupdate(den_v.at[r_e, pl.ds(L, L)], wc1)
            for b in range(B):
                wks = [plsc.load_gather(w_v, [_splat(b * NH + h), _splat(k)])
                       for h in range(NH)]
                for c in range(F // L):
                    plsc.addupdate(
                        acc_v.at[b, r_e, pl.ds(c * L, L)],
                        wks[c // 2] * hrow_v[b * L + k, pl.ds(c * L, L)])
            return 0

        lax.fori_loop(0, jnp.minimum(total - win * L, L), edge, 0)
        return 0

    lax.fori_loop(0, nwin, winB, 0)

    for b in range(B):
        pltpu.sync_copy(acc_v.at[b], acc_hbm.at[b, pl.ds(base, RPW)])
    pltpu.sync_copy(den_v, den_hbm.at[pl.ds(base, RPW)])


def _blockdiag(a):
    heads, dim = a.shape
    eye = jnp.eye(heads, dtype=a.dtype)
    return (a[:, :, None] * eye[:, None, :]).reshape(heads * dim, heads)


def kernel(x, embedding, W_proj, b_proj, W1, a1_src, a1_dst, b1, W2,
           a2_src, a2_dst, b2):
    adj = pl.pallas_call(
        _adj_body,
        out_shape=jax.ShapeDtypeStruct((N, N), jnp.float32),
    )(embedding)

    cols, counts = _scan_kernel(adj)

    w1r = W1.reshape(F, F)
    w2r = W2.reshape(F, F)
    a1s = _blockdiag(a1_src)
    a1d = _blockdiag(a1_dst)
    a2s = _blockdiag(a2_src)
    a2d = _blockdiag(a2_dst)
    bp2 = b_proj.reshape(1, F)
    b1r = b1.reshape(1, F)
    b2r = b2.reshape(1, F)
    hexp = _blockdiag(jnp.ones((NH, F // NH), jnp.float32)).T

    full = lambda shape: pl.BlockSpec(shape, lambda b: (0,) * len(shape))
    batched = lambda shape: pl.BlockSpec((1,) + shape,
                                         lambda b: (b,) + (0,) * len(shape))
    h1, st1, d1 = pl.pallas_call(
        _proj1_body,
        grid=(B,),
        in_specs=[batched((N, F)), full((F, F)), full((1, F)), full((F, F)),
                  full((F, NH)), full((F, NH))],
        out_specs=[batched((N, F)), batched((NH, N)), batched((N, NH))],
        out_shape=[
            jax.ShapeDtypeStruct((B, N, F), jnp.float32),
            jax.ShapeDtypeStruct((B, NH, N), jnp.float32),
            jax.ShapeDtypeStruct((B, N, NH), jnp.float32),
        ],
    )(x, W_proj, bp2, w1r, a1s, a1d)

    zstrip = jnp.zeros((B, RPW, F), jnp.float32)
    acc1, den1 = _agg_kernel(h1.reshape(B * N, F), st1.reshape(B * NH * N),
                             d1, cols, counts, zstrip)

    h2, st2, d2 = pl.pallas_call(
        _proj2_body,
        grid=(B,),
        in_specs=[batched((N, F)), batched((NH, N)), full((NH, F)),
                  full((1, F)), full((F, F)), full((F, NH)), full((F, NH))],
        out_specs=[batched((N, F)), batched((NH, N)), batched((N, NH))],
        out_shape=[
            jax.ShapeDtypeStruct((B, N, F), jnp.float32),
            jax.ShapeDtypeStruct((B, NH, N), jnp.float32),
            jax.ShapeDtypeStruct((B, N, NH), jnp.float32),
        ],
    )(acc1, jnp.transpose(den1.reshape(N, B, NH), (1, 2, 0)), hexp, b1r,
      w2r, a2s, a2d)

    acc2, den2 = _agg_kernel(h2.reshape(B * N, F), st2.reshape(B * NH * N),
                             d2, cols, counts, zstrip)

    out = pl.pallas_call(
        _final_body,
        grid=(B,),
        in_specs=[batched((N, F)), batched((NH, N)), full((NH, F)),
                  full((1, F))],
        out_specs=batched((N, F)),
        out_shape=jax.ShapeDtypeStruct((B, N, F), jnp.float32),
    )(acc2, jnp.transpose(den2.reshape(N, B, NH), (1, 2, 0)), hexp, b2r)
    return out
